# baseline (speedup 1.0000x reference)
import jax
import jax.numpy as jnp
from jax import lax
from jax.experimental import pallas as pl
from jax.experimental.pallas import tpu as pltpu
from jax.experimental.pallas import tpu_sc as plsc

N = 10000
E = 160000
F = 128
EPS = 1e-08
CLIP = 100.0

_NPASS = 5
_NR = 1024
_ACC_ROWS = 1152
_WSLICE = _NR // 16
_SLICE_E = E // 16
_BS = 2000
_NBLK = _SLICE_E // _BS
_CGRP = _BS // 16
_CAP = 2064
_OUT_ROWS = 10 * _NR


def _mlp1_body(s_ref, v_ref, w1_ref, b1_ref, w2_ref, b2_ref, o_ref):
    x = jnp.dot(s_ref[...], w1_ref[...], preferred_element_type=jnp.float32) + b1_ref[...]
    x = x * jax.nn.sigmoid(x)
    o_ref[:, :3 * F] = jnp.dot(x, w2_ref[...], preferred_element_type=jnp.float32) + b2_ref[...]
    o_ref[:, 3 * F:] = v_ref[...]


def _phase1(s2, v2, W1, b1, W2, b2):
    BN = 1000
    return pl.pallas_call(
        _mlp1_body,
        grid=(N // BN,),
        in_specs=[
            pl.BlockSpec((BN, F), lambda i: (i, 0)),
            pl.BlockSpec((BN, 3 * F), lambda i: (i, 0)),
            pl.BlockSpec((F, F), lambda i: (0, 0)),
            pl.BlockSpec((1, F), lambda i: (0, 0)),
            pl.BlockSpec((F, 3 * F), lambda i: (0, 0)),
            pl.BlockSpec((1, 3 * F), lambda i: (0, 0)),
        ],
        out_specs=pl.BlockSpec((BN, 6 * F), lambda i: (i, 0)),
        out_shape=jax.ShapeDtypeStruct((N, 6 * F), jnp.float32),
    )(s2, v2, W1, b1.reshape(1, F), W2, b2.reshape(1, 3 * F))


def _mix_body(s_ref, v_ref, a0_ref, a1_ref, a2_ref, a3_ref,
              wv_ref, wm1_ref, bm1_ref, wm2_ref, bm2_ref,
              so_ref, vo_ref):
    sb = s_ref[...]
    vb = v_ref[...]
    dva = (a1_ref, a2_ref, a3_ref)
    s1 = sb + jnp.clip(a0_ref[...], -CLIP, CLIP)
    va = [vb[:, a * F:(a + 1) * F] + jnp.clip(dva[a][...], -CLIP, CLIP)
          for a in range(3)]
    wv = wv_ref[...]
    fa = [jnp.dot(va[a], wv, preferred_element_type=jnp.float32) for a in range(3)]
    vl = [fa[a][:, :F] for a in range(3)]
    vr = [fa[a][:, F:] for a in range(3)]
    vn = jnp.sqrt(vr[0] ** 2 + vr[1] ** 2 + vr[2] ** 2 + EPS)
    wm1 = wm1_ref[...]
    h1 = (jnp.dot(s1, wm1[:F, :], preferred_element_type=jnp.float32)
          + jnp.dot(vn, wm1[F:, :], preferred_element_type=jnp.float32) + bm1_ref[...])
    h1 = h1 * jax.nn.sigmoid(h1)
    h = jnp.dot(h1, wm2_ref[...], preferred_element_type=jnp.float32) + bm2_ref[...]
    dsh = h[:, :F]
    dvh = h[:, F:2 * F]
    dsvh = h[:, 2 * F:]
    dot3 = vr[0] * vl[0] + vr[1] * vl[1] + vr[2] * vl[2]
    so_ref[...] = s1 + jnp.clip(dsh + dsvh * dot3, -CLIP, CLIP)
    for a in range(3):
        vo_ref[:, a * F:(a + 1) * F] = va[a] + jnp.clip(vl[a] * dvh, -CLIP, CLIP)


def _phase3(s2, v2, acc, Wv, Wm1, bm1, Wm2, bm2):
    BN = 1000
    return pl.pallas_call(
        _mix_body,
        grid=(N // BN,),
        in_specs=[
            pl.BlockSpec((BN, F), lambda i: (i, 0)),
            pl.BlockSpec((BN, 3 * F), lambda i: (i, 0)),
            pl.BlockSpec((BN, F), lambda i: (i, 0)),
            pl.BlockSpec((BN, F), lambda i: (i, 0)),
            pl.BlockSpec((BN, F), lambda i: (i, 0)),
            pl.BlockSpec((BN, F), lambda i: (i, 0)),
            pl.BlockSpec((F, 2 * F), lambda i: (0, 0)),
            pl.BlockSpec((2 * F, F), lambda i: (0, 0)),
            pl.BlockSpec((1, F), lambda i: (0, 0)),
            pl.BlockSpec((F, 3 * F), lambda i: (0, 0)),
            pl.BlockSpec((1, 3 * F), lambda i: (0, 0)),
        ],
        out_specs=[
            pl.BlockSpec((BN, F), lambda i: (i, 0)),
            pl.BlockSpec((BN, 3 * F), lambda i: (i, 0)),
        ],
        out_shape=[
            jax.ShapeDtypeStruct((N, F), jnp.float32),
            jax.ShapeDtypeStruct((N, 3 * F), jnp.float32),
        ],
    )(s2, v2, *acc, Wv, Wm1, bm1.reshape(1, F), Wm2, bm2.reshape(1, 3 * F))


def _edge_body(xv_hbm, wij_hbm, dir_hbm, snd_hbm, rcv_hbm,
               out0_hbm, out1_hbm, out2_hbm, out3_hbm,
               snd_b0, snd_b1, rcv_b0, rcv_b1, dir_b0, dir_b1,
               snd_c, rcv_c, eidx_c, d0_c, d1_c, d2_c,
               wijA, xvA, wijB, xvB,
               mA0, mA1, mA2, mA3, mB0, mB1, mB2, mB3,
               sem_b, sem_gA, sem_gB, sem_sA, sem_sB, sem_z,
               acc0, acc1, acc2, acc3):
    outs = (out0_hbm, out1_hbm, out2_hbm, out3_hbm)
    accs = (acc0, acc1, acc2, acc3)
    msgsA = (mA0, mA1, mA2, mA3)
    msgsB = (mB0, mB1, mB2, mB3)
    stage = ((snd_b0, rcv_b0, dir_b0), (snd_b1, rcv_b1, dir_b1))
    cid = lax.axis_index("c")
    sid = lax.axis_index("s")
    iota16 = lax.iota(jnp.int32, 16)
    zero16f = jnp.zeros((16,), jnp.float32)
    zero16i = jnp.zeros((16,), jnp.int32)

    def _stage_copies(b):
        off = sid * _SLICE_E + b * _BS
        sb, rb, db = stage[b % 2]
        return (
            pltpu.make_async_copy(snd_hbm.at[pl.ds(off, _BS)], sb, sem_b),
            pltpu.make_async_copy(rcv_hbm.at[pl.ds(off, _BS)], rb, sem_b),
            pltpu.make_async_copy(dir_hbm.at[pl.ds(off * 3, 3 * _BS)], db, sem_b),
        )

    def _stage_fire(b):
        for c in _stage_copies(b):
            c.start()

    def _stage_wait(b):
        for c in _stage_copies(b):
            c.wait()

    def _compute_half(bufs, msgs, base):
        wij, xv = bufs

        def _one_edge(i):
            ei = jnp.full((16,), base + i, jnp.int32)
            d0 = plsc.load_gather(d0_c, [ei])
            d1 = plsc.load_gather(d1_c, [ei])
            d2 = plsc.load_gather(d2_c, [ei])
            for fb in range(8):
                o = fb * 16
                m0 = wij[i, pl.ds(o, 16)] * xv[i, pl.ds(o, 16)]
                msgs[0][i, pl.ds(o, 16)] = m0
                m1 = wij[i, pl.ds(F + o, 16)] * xv[i, pl.ds(F + o, 16)]
                m2 = wij[i, pl.ds(2 * F + o, 16)] * xv[i, pl.ds(2 * F + o, 16)]
                v0 = xv[i, pl.ds(3 * F + o, 16)]
                v1 = xv[i, pl.ds(4 * F + o, 16)]
                v2v = xv[i, pl.ds(5 * F + o, 16)]
                msgs[1][i, pl.ds(o, 16)] = m1 * d0 + m2 * v0
                msgs[2][i, pl.ds(o, 16)] = m1 * d1 + m2 * v1
                msgs[3][i, pl.ds(o, 16)] = m1 * d2 + m2 * v2v

        def _edge(j, c2):
            _one_edge(2 * j)
            _one_edge(2 * j + 1)
            return c2
        lax.fori_loop(0, 8, _edge, 0)

    def _gather_copies(bufs, sem, base):
        wij, xv = bufs
        ei = eidx_c[pl.ds(base, 16)]
        ri = rcv_c[pl.ds(base, 16)]
        return (
            pltpu.make_async_copy(wij_hbm.at[ei], wij, sem),
            pltpu.make_async_copy(xv_hbm.at[ri], xv, sem),
        )

    def _scatter_copies(msgs, sem, base):
        si = snd_c[pl.ds(base, 16)]
        return tuple(
            pltpu.make_async_copy(msgs[q], accs[q].at[si], sem)
            for q in range(4)
        )

    def _run_pairs(npairs):
        @pl.when(npairs > 0)
        def _():
            for c in _gather_copies((wijA, xvA), sem_gA, 0):
                c.start()

        def _pair(h, c):
            base = h * 32
            for cp in _gather_copies((wijB, xvB), sem_gB, base + 16):
                cp.start()
            for cp in _gather_copies((wijA, xvA), sem_gA, base):
                cp.wait()

            @pl.when(h > 0)
            def _():
                for cp in _scatter_copies(msgsA, sem_sA, base):
                    cp.wait()
            _compute_half((wijA, xvA), msgsA, base)
            for cp in _scatter_copies(msgsA, sem_sA, base):
                cp.start(add=True)

            @pl.when(h + 1 < npairs)
            def _():
                for cp in _gather_copies((wijA, xvA), sem_gA, base + 32):
                    cp.start()
            for cp in _gather_copies((wijB, xvB), sem_gB, base + 16):
                cp.wait()

            @pl.when(h > 0)
            def _():
                for cp in _scatter_copies(msgsB, sem_sB, base):
                    cp.wait()
            _compute_half((wijB, xvB), msgsB, base + 16)
            for cp in _scatter_copies(msgsB, sem_sB, base + 16):
                cp.start(add=True)
            return c
        lax.fori_loop(0, npairs, _pair, 0)

        @pl.when(npairs > 0)
        def _():
            for cp in _scatter_copies(msgsA, sem_sA, 0):
                cp.wait()
            for cp in _scatter_copies(msgsB, sem_sB, 0):
                cp.wait()

    def _pass_body(p, carry):
        rng = 2 * p + cid
        lo = rng * _NR

        def _zm(i, c):
            for fb in range(8):
                mA0[i, pl.ds(fb * 16, 16)] = zero16f
            return c
        lax.fori_loop(0, 16, _zm, 0)
        zcps = []
        for q in range(4):
            for j in range(4):
                zcps.append(pltpu.make_async_copy(
                    mA0, accs[q].at[pl.ds(sid * _WSLICE + j * 16, 16)], sem_z))
        for cp in zcps:
            cp.start()
        for cp in zcps:
            cp.wait()
        plsc.subcore_barrier()

        def _comp_body(k, wp, par, ebase):
            sb, rb, db = stage[par]
            sv = sb[pl.ds(k * 16, 16)]
            rv = rb[pl.ds(k * 16, 16)]
            rel = sv - lo
            mask = (rel >= 0) & (rel < _NR)
            csum = plsc.cumsum(mask.astype(jnp.int32))
            pos = jnp.where(mask, wp + csum - 1, _CAP - 1)
            plsc.store_scatter(snd_c, [pos], rel)
            plsc.store_scatter(rcv_c, [pos], rv)
            ev = jnp.full((16,), k * 16, jnp.int32) + iota16
            plsc.store_scatter(eidx_c, [pos], ev + ebase)
            kidx3 = ev * 3
            plsc.store_scatter(d0_c, [pos], plsc.load_gather(db, [kidx3]))
            plsc.store_scatter(d1_c, [pos], plsc.load_gather(db, [kidx3 + 1]))
            plsc.store_scatter(d2_c, [pos], plsc.load_gather(db, [kidx3 + 2]))
            return wp + jnp.max(csum)

        _stage_fire(0)
        wp = 0
        for b in range(_NBLK):
            _stage_wait(b)
            if b + 1 < _NBLK:
                _stage_fire(b + 1)
            off = sid * _SLICE_E + b * _BS
            wp = lax.fori_loop(
                0, _CGRP, lambda k, w: _comp_body(k, w, b % 2, off), wp)
            npairs = wp // 32
            _run_pairs(npairs)
            rem = wp - npairs * 32
            for arr in (snd_c, rcv_c, eidx_c, d0_c, d1_c, d2_c):
                t0 = arr[pl.ds(npairs * 32, 16)]
                t1 = arr[pl.ds(npairs * 32 + 16, 16)]
                arr[pl.ds(0, 16)] = t0
                arr[pl.ds(16, 16)] = t1
            wp = rem

        dummy = jnp.full((16,), _NR, jnp.int32)
        snd_c[pl.ds(wp, 16)] = dummy
        snd_c[pl.ds(wp + 16, 16)] = dummy
        for arr in (rcv_c, eidx_c):
            arr[pl.ds(wp, 16)] = zero16i
            arr[pl.ds(wp + 16, 16)] = zero16i
        for arr in (d0_c, d1_c, d2_c):
            arr[pl.ds(wp, 16)] = zero16f
            arr[pl.ds(wp + 16, 16)] = zero16f
        _run_pairs((wp + 31) // 32)

        plsc.subcore_barrier()

        wcps = []
        for q in range(4):
            wcps.append(pltpu.make_async_copy(
                accs[q].at[pl.ds(sid * _WSLICE, _WSLICE)],
                outs[q].at[pl.ds(lo + sid * _WSLICE, _WSLICE)], sem_z))
        for cp in wcps:
            cp.start()
        for cp in wcps:
            cp.wait()
        return carry

    lax.fori_loop(0, _NPASS, _pass_body, 0)


def _edge_phase(xv, wij2, dirf, senders, receivers):
    mesh = plsc.VectorSubcoreMesh(core_axis_name="c", subcore_axis_name="s")
    fn = pl.kernel(
        _edge_body,
        out_type=[jax.ShapeDtypeStruct((_OUT_ROWS, F), jnp.float32)] * 4,
        mesh=mesh,
        compiler_params=pltpu.CompilerParams(needs_layout_passes=False),
        scratch_types=[
            pltpu.VMEM((_BS,), jnp.int32),
            pltpu.VMEM((_BS,), jnp.int32),
            pltpu.VMEM((_BS,), jnp.int32),
            pltpu.VMEM((_BS,), jnp.int32),
            pltpu.VMEM((3 * _BS,), jnp.float32),
            pltpu.VMEM((3 * _BS,), jnp.float32),
            pltpu.VMEM((_CAP,), jnp.int32),
            pltpu.VMEM((_CAP,), jnp.int32),
            pltpu.VMEM((_CAP,), jnp.int32),
            pltpu.VMEM((_CAP,), jnp.float32),
            pltpu.VMEM((_CAP,), jnp.float32),
            pltpu.VMEM((_CAP,), jnp.float32),
            pltpu.VMEM((16, 3 * F), jnp.float32),
            pltpu.VMEM((16, 6 * F), jnp.float32),
            pltpu.VMEM((16, 3 * F), jnp.float32),
            pltpu.VMEM((16, 6 * F), jnp.float32),
            pltpu.VMEM((16, F), jnp.float32),
            pltpu.VMEM((16, F), jnp.float32),
            pltpu.VMEM((16, F), jnp.float32),
            pltpu.VMEM((16, F), jnp.float32),
            pltpu.VMEM((16, F), jnp.float32),
            pltpu.VMEM((16, F), jnp.float32),
            pltpu.VMEM((16, F), jnp.float32),
            pltpu.VMEM((16, F), jnp.float32),
            pltpu.SemaphoreType.DMA,
            pltpu.SemaphoreType.DMA,
            pltpu.SemaphoreType.DMA,
            pltpu.SemaphoreType.DMA,
            pltpu.SemaphoreType.DMA,
            pltpu.SemaphoreType.DMA,
            pltpu.VMEM_SHARED((_ACC_ROWS, F), jnp.float32),
            pltpu.VMEM_SHARED((_ACC_ROWS, F), jnp.float32),
            pltpu.VMEM_SHARED((_ACC_ROWS, F), jnp.float32),
            pltpu.VMEM_SHARED((_ACC_ROWS, F), jnp.float32),
        ],
    )
    return fn(xv, wij2, dirf, senders, receivers)


def kernel(s, v, dir_ij, Wij, senders, receivers,
           W1, b1, W2, b2, Wm1, bm1, Wm2, bm2, Wv):
    s2 = s.reshape(N, F)
    v2 = v.reshape(N, 3 * F)
    wij2 = Wij.reshape(E, 3 * F)
    dirf = dir_ij.reshape(3 * E)
    xv = _phase1(s2, v2, W1, b1, W2, b2)
    acc = _edge_phase(xv, wij2, dirf, senders, receivers)
    so, vo = _phase3(s2, v2, acc, Wv, Wm1, bm1, Wm2, bm2)
    return so.reshape(N, 1, F), vo.reshape(N, 3, F)


# speedup vs baseline: 1.0501x; 1.0501x over previous
import jax
import jax.numpy as jnp
from jax import lax
from jax.experimental import pallas as pl
from jax.experimental.pallas import tpu as pltpu
from jax.experimental.pallas import tpu_sc as plsc

N = 10000
E = 160000
F = 128
EPS = 1e-08
CLIP = 100.0

_NPASS = 4
_NR = 1280
_ACC_ROWS = 1296
_WSLICE = _NR // 16
_SLICE_E = E // 16
_BS = 2000
_NBLK = _SLICE_E // _BS
_CGRP = _BS // 16
_CAP = 2064
_OUT_ROWS = 8 * _NR


def _mlp1_body(s_ref, v_ref, w1_ref, b1_ref, w2_ref, b2_ref, o_ref):
    x = jnp.dot(s_ref[...], w1_ref[...], preferred_element_type=jnp.float32) + b1_ref[...]
    x = x * jax.nn.sigmoid(x)
    o_ref[:, :3 * F] = jnp.dot(x, w2_ref[...], preferred_element_type=jnp.float32) + b2_ref[...]
    o_ref[:, 3 * F:] = v_ref[...]


def _phase1(s2, v2, W1, b1, W2, b2):
    BN = 1000
    return pl.pallas_call(
        _mlp1_body,
        grid=(N // BN,),
        in_specs=[
            pl.BlockSpec((BN, F), lambda i: (i, 0)),
            pl.BlockSpec((BN, 3 * F), lambda i: (i, 0)),
            pl.BlockSpec((F, F), lambda i: (0, 0)),
            pl.BlockSpec((1, F), lambda i: (0, 0)),
            pl.BlockSpec((F, 3 * F), lambda i: (0, 0)),
            pl.BlockSpec((1, 3 * F), lambda i: (0, 0)),
        ],
        out_specs=pl.BlockSpec((BN, 6 * F), lambda i: (i, 0)),
        out_shape=jax.ShapeDtypeStruct((N, 6 * F), jnp.float32),
    )(s2, v2, W1, b1.reshape(1, F), W2, b2.reshape(1, 3 * F))


def _mix_body(s_ref, v_ref, a0_ref, a1_ref, a2_ref, a3_ref,
              wv_ref, wm1_ref, bm1_ref, wm2_ref, bm2_ref,
              so_ref, vo_ref):
    sb = s_ref[...]
    vb = v_ref[...]
    dva = (a1_ref, a2_ref, a3_ref)
    s1 = sb + jnp.clip(a0_ref[...], -CLIP, CLIP)
    va = [vb[:, a * F:(a + 1) * F] + jnp.clip(dva[a][...], -CLIP, CLIP)
          for a in range(3)]
    wv = wv_ref[...]
    fa = [jnp.dot(va[a], wv, preferred_element_type=jnp.float32) for a in range(3)]
    vl = [fa[a][:, :F] for a in range(3)]
    vr = [fa[a][:, F:] for a in range(3)]
    vn = jnp.sqrt(vr[0] ** 2 + vr[1] ** 2 + vr[2] ** 2 + EPS)
    wm1 = wm1_ref[...]
    h1 = (jnp.dot(s1, wm1[:F, :], preferred_element_type=jnp.float32)
          + jnp.dot(vn, wm1[F:, :], preferred_element_type=jnp.float32) + bm1_ref[...])
    h1 = h1 * jax.nn.sigmoid(h1)
    h = jnp.dot(h1, wm2_ref[...], preferred_element_type=jnp.float32) + bm2_ref[...]
    dsh = h[:, :F]
    dvh = h[:, F:2 * F]
    dsvh = h[:, 2 * F:]
    dot3 = vr[0] * vl[0] + vr[1] * vl[1] + vr[2] * vl[2]
    so_ref[...] = s1 + jnp.clip(dsh + dsvh * dot3, -CLIP, CLIP)
    for a in range(3):
        vo_ref[:, a * F:(a + 1) * F] = va[a] + jnp.clip(vl[a] * dvh, -CLIP, CLIP)


def _phase3(s2, v2, acc, Wv, Wm1, bm1, Wm2, bm2):
    BN = 1000
    return pl.pallas_call(
        _mix_body,
        grid=(N // BN,),
        in_specs=[
            pl.BlockSpec((BN, F), lambda i: (i, 0)),
            pl.BlockSpec((BN, 3 * F), lambda i: (i, 0)),
            pl.BlockSpec((BN, F), lambda i: (i, 0)),
            pl.BlockSpec((BN, F), lambda i: (i, 0)),
            pl.BlockSpec((BN, F), lambda i: (i, 0)),
            pl.BlockSpec((BN, F), lambda i: (i, 0)),
            pl.BlockSpec((F, 2 * F), lambda i: (0, 0)),
            pl.BlockSpec((2 * F, F), lambda i: (0, 0)),
            pl.BlockSpec((1, F), lambda i: (0, 0)),
            pl.BlockSpec((F, 3 * F), lambda i: (0, 0)),
            pl.BlockSpec((1, 3 * F), lambda i: (0, 0)),
        ],
        out_specs=[
            pl.BlockSpec((BN, F), lambda i: (i, 0)),
            pl.BlockSpec((BN, 3 * F), lambda i: (i, 0)),
        ],
        out_shape=[
            jax.ShapeDtypeStruct((N, F), jnp.float32),
            jax.ShapeDtypeStruct((N, 3 * F), jnp.float32),
        ],
    )(s2, v2, *acc, Wv, Wm1, bm1.reshape(1, F), Wm2, bm2.reshape(1, 3 * F))


def _edge_body(xv_hbm, wij_hbm, dir_hbm, snd_hbm, rcv_hbm,
               out0_hbm, out1_hbm, out2_hbm, out3_hbm,
               snd_b0, snd_b1, rcv_b0, rcv_b1, dir_b0, dir_b1,
               snd_c, rcv_c, eidx_c, d0_c, d1_c, d2_c,
               wijA, xvA, wijB, xvB,
               mA0, mA1, mA2, mA3, mB0, mB1, mB2, mB3,
               sem_b, sem_gA, sem_gB, sem_sA, sem_sB, sem_z,
               acc0, acc1, acc2, acc3):
    outs = (out0_hbm, out1_hbm, out2_hbm, out3_hbm)
    accs = (acc0, acc1, acc2, acc3)
    msgsA = (mA0, mA1, mA2, mA3)
    msgsB = (mB0, mB1, mB2, mB3)
    stage = ((snd_b0, rcv_b0, dir_b0), (snd_b1, rcv_b1, dir_b1))
    cid = lax.axis_index("c")
    sid = lax.axis_index("s")
    iota16 = lax.iota(jnp.int32, 16)
    zero16f = jnp.zeros((16,), jnp.float32)
    zero16i = jnp.zeros((16,), jnp.int32)

    def _stage_copies(b):
        off = sid * _SLICE_E + b * _BS
        sb, rb, db = stage[b % 2]
        return (
            pltpu.make_async_copy(snd_hbm.at[pl.ds(off, _BS)], sb, sem_b),
            pltpu.make_async_copy(rcv_hbm.at[pl.ds(off, _BS)], rb, sem_b),
            pltpu.make_async_copy(dir_hbm.at[pl.ds(off * 3, 3 * _BS)], db, sem_b),
        )

    def _stage_fire(b):
        for c in _stage_copies(b):
            c.start()

    def _stage_wait(b):
        for c in _stage_copies(b):
            c.wait()

    def _compute_half(bufs, msgs, base):
        wij, xv = bufs

        def _one_edge(i):
            ei = jnp.full((16,), base + i, jnp.int32)
            d0 = plsc.load_gather(d0_c, [ei])
            d1 = plsc.load_gather(d1_c, [ei])
            d2 = plsc.load_gather(d2_c, [ei])
            for fb in range(8):
                o = fb * 16
                m0 = wij[i, pl.ds(o, 16)] * xv[i, pl.ds(o, 16)]
                msgs[0][i, pl.ds(o, 16)] = m0
                m1 = wij[i, pl.ds(F + o, 16)] * xv[i, pl.ds(F + o, 16)]
                m2 = wij[i, pl.ds(2 * F + o, 16)] * xv[i, pl.ds(2 * F + o, 16)]
                v0 = xv[i, pl.ds(3 * F + o, 16)]
                v1 = xv[i, pl.ds(4 * F + o, 16)]
                v2v = xv[i, pl.ds(5 * F + o, 16)]
                msgs[1][i, pl.ds(o, 16)] = m1 * d0 + m2 * v0
                msgs[2][i, pl.ds(o, 16)] = m1 * d1 + m2 * v1
                msgs[3][i, pl.ds(o, 16)] = m1 * d2 + m2 * v2v

        def _edge(j, c2):
            _one_edge(j)
            return c2
        lax.fori_loop(0, 16, _edge, 0)

    def _gather_copies(bufs, sem, base):
        wij, xv = bufs
        ei = eidx_c[pl.ds(base, 16)]
        ri = rcv_c[pl.ds(base, 16)]
        return (
            pltpu.make_async_copy(wij_hbm.at[ei], wij, sem),
            pltpu.make_async_copy(xv_hbm.at[ri], xv, sem),
        )

    def _scatter_copies(msgs, sem, base):
        si = snd_c[pl.ds(base, 16)]
        return tuple(
            pltpu.make_async_copy(msgs[q], accs[q].at[si], sem)
            for q in range(4)
        )

    def _run_pairs(npairs):
        @pl.when(npairs > 0)
        def _():
            for c in _gather_copies((wijA, xvA), sem_gA, 0):
                c.start()

        def _pair(h, c):
            base = h * 32
            for cp in _gather_copies((wijB, xvB), sem_gB, base + 16):
                cp.start()
            for cp in _gather_copies((wijA, xvA), sem_gA, base):
                cp.wait()

            @pl.when(h > 0)
            def _():
                for cp in _scatter_copies(msgsA, sem_sA, base):
                    cp.wait()
            _compute_half((wijA, xvA), msgsA, base)
            for cp in _scatter_copies(msgsA, sem_sA, base):
                cp.start(add=True)

            @pl.when(h + 1 < npairs)
            def _():
                for cp in _gather_copies((wijA, xvA), sem_gA, base + 32):
                    cp.start()
            for cp in _gather_copies((wijB, xvB), sem_gB, base + 16):
                cp.wait()

            @pl.when(h > 0)
            def _():
                for cp in _scatter_copies(msgsB, sem_sB, base):
                    cp.wait()
            _compute_half((wijB, xvB), msgsB, base + 16)
            for cp in _scatter_copies(msgsB, sem_sB, base + 16):
                cp.start(add=True)
            return c
        lax.fori_loop(0, npairs, _pair, 0)

        @pl.when(npairs > 0)
        def _():
            for cp in _scatter_copies(msgsA, sem_sA, 0):
                cp.wait()
            for cp in _scatter_copies(msgsB, sem_sB, 0):
                cp.wait()

    def _pass_body(p, carry):
        rng = 2 * p + cid
        lo = rng * _NR

        def _zm(i, c):
            for fb in range(8):
                mA0[i, pl.ds(fb * 16, 16)] = zero16f
            return c
        lax.fori_loop(0, 16, _zm, 0)
        zcps = []
        for q in range(4):
            for j in range(_WSLICE // 16):
                zcps.append(pltpu.make_async_copy(
                    mA0, accs[q].at[pl.ds(sid * _WSLICE + j * 16, 16)], sem_z))
        for cp in zcps:
            cp.start()
        for cp in zcps:
            cp.wait()
        plsc.subcore_barrier()

        def _comp_body(k, wp, par, ebase):
            sb, rb, db = stage[par]
            sv = sb[pl.ds(k * 16, 16)]
            rv = rb[pl.ds(k * 16, 16)]
            rel = sv - lo
            mask = (rel >= 0) & (rel < _NR)
            csum = plsc.cumsum(mask.astype(jnp.int32))
            pos = jnp.where(mask, wp + csum - 1, _CAP - 1)
            plsc.store_scatter(snd_c, [pos], rel)
            plsc.store_scatter(rcv_c, [pos], rv)
            ev = jnp.full((16,), k * 16, jnp.int32) + iota16
            plsc.store_scatter(eidx_c, [pos], ev + ebase)
            kidx3 = ev * 3
            plsc.store_scatter(d0_c, [pos], plsc.load_gather(db, [kidx3]))
            plsc.store_scatter(d1_c, [pos], plsc.load_gather(db, [kidx3 + 1]))
            plsc.store_scatter(d2_c, [pos], plsc.load_gather(db, [kidx3 + 2]))
            return wp + jnp.max(csum)

        _stage_fire(0)
        wp = 0
        for b in range(_NBLK):
            _stage_wait(b)
            if b + 1 < _NBLK:
                _stage_fire(b + 1)
            off = sid * _SLICE_E + b * _BS
            wp = lax.fori_loop(
                0, _CGRP, lambda k, w: _comp_body(k, w, b % 2, off), wp)
            npairs = wp // 32
            _run_pairs(npairs)
            rem = wp - npairs * 32
            for arr in (snd_c, rcv_c, eidx_c, d0_c, d1_c, d2_c):
                t0 = arr[pl.ds(npairs * 32, 16)]
                t1 = arr[pl.ds(npairs * 32 + 16, 16)]
                arr[pl.ds(0, 16)] = t0
                arr[pl.ds(16, 16)] = t1
            wp = rem

        dummy = jnp.full((16,), _NR, jnp.int32)
        snd_c[pl.ds(wp, 16)] = dummy
        snd_c[pl.ds(wp + 16, 16)] = dummy
        for arr in (rcv_c, eidx_c):
            arr[pl.ds(wp, 16)] = zero16i
            arr[pl.ds(wp + 16, 16)] = zero16i
        for arr in (d0_c, d1_c, d2_c):
            arr[pl.ds(wp, 16)] = zero16f
            arr[pl.ds(wp + 16, 16)] = zero16f
        _run_pairs((wp + 31) // 32)

        plsc.subcore_barrier()

        wcps = []
        for q in range(4):
            wcps.append(pltpu.make_async_copy(
                accs[q].at[pl.ds(sid * _WSLICE, _WSLICE)],
                outs[q].at[pl.ds(lo + sid * _WSLICE, _WSLICE)], sem_z))
        for cp in wcps:
            cp.start()
        for cp in wcps:
            cp.wait()
        return carry

    lax.fori_loop(0, _NPASS, _pass_body, 0)


def _edge_phase(xv, wij2, dirf, senders, receivers):
    mesh = plsc.VectorSubcoreMesh(core_axis_name="c", subcore_axis_name="s")
    fn = pl.kernel(
        _edge_body,
        out_type=[jax.ShapeDtypeStruct((_OUT_ROWS, F), jnp.float32)] * 4,
        mesh=mesh,
        compiler_params=pltpu.CompilerParams(needs_layout_passes=False),
        scratch_types=[
            pltpu.VMEM((_BS,), jnp.int32),
            pltpu.VMEM((_BS,), jnp.int32),
            pltpu.VMEM((_BS,), jnp.int32),
            pltpu.VMEM((_BS,), jnp.int32),
            pltpu.VMEM((3 * _BS,), jnp.float32),
            pltpu.VMEM((3 * _BS,), jnp.float32),
            pltpu.VMEM((_CAP,), jnp.int32),
            pltpu.VMEM((_CAP,), jnp.int32),
            pltpu.VMEM((_CAP,), jnp.int32),
            pltpu.VMEM((_CAP,), jnp.float32),
            pltpu.VMEM((_CAP,), jnp.float32),
            pltpu.VMEM((_CAP,), jnp.float32),
            pltpu.VMEM((16, 3 * F), jnp.float32),
            pltpu.VMEM((16, 6 * F), jnp.float32),
            pltpu.VMEM((16, 3 * F), jnp.float32),
            pltpu.VMEM((16, 6 * F), jnp.float32),
            pltpu.VMEM((16, F), jnp.float32),
            pltpu.VMEM((16, F), jnp.float32),
            pltpu.VMEM((16, F), jnp.float32),
            pltpu.VMEM((16, F), jnp.float32),
            pltpu.VMEM((16, F), jnp.float32),
            pltpu.VMEM((16, F), jnp.float32),
            pltpu.VMEM((16, F), jnp.float32),
            pltpu.VMEM((16, F), jnp.float32),
            pltpu.SemaphoreType.DMA,
            pltpu.SemaphoreType.DMA,
            pltpu.SemaphoreType.DMA,
            pltpu.SemaphoreType.DMA,
            pltpu.SemaphoreType.DMA,
            pltpu.SemaphoreType.DMA,
            pltpu.VMEM_SHARED((_ACC_ROWS, F), jnp.float32),
            pltpu.VMEM_SHARED((_ACC_ROWS, F), jnp.float32),
            pltpu.VMEM_SHARED((_ACC_ROWS, F), jnp.float32),
            pltpu.VMEM_SHARED((_ACC_ROWS, F), jnp.float32),
        ],
    )
    return fn(xv, wij2, dirf, senders, receivers)


def kernel(s, v, dir_ij, Wij, senders, receivers,
           W1, b1, W2, b2, Wm1, bm1, Wm2, bm2, Wv):
    s2 = s.reshape(N, F)
    v2 = v.reshape(N, 3 * F)
    wij2 = Wij.reshape(E, 3 * F)
    dirf = dir_ij.reshape(3 * E)
    xv = _phase1(s2, v2, W1, b1, W2, b2)
    acc = _edge_phase(xv, wij2, dirf, senders, receivers)
    so, vo = _phase3(s2, v2, acc, Wv, Wm1, bm1, Wm2, bm2)
    return so.reshape(N, 1, F), vo.reshape(N, 3, F)
